# baseline (speedup 1.0000x reference)
import jax
import jax.numpy as jnp
from jax import lax
from jax.experimental import pallas as pl
from jax.experimental.pallas import tpu as pltpu
from jax.experimental.pallas import tpu_sc as plsc

B = 16384
EMB = 64
NC = 2
NS = 16
NW = NC * NS
CHUNK = 128
CHUNKS = B // NW // CHUNK
PER_W = CHUNKS * CHUNK
LANES = 16
UI_K = 5
SW_K = 4


def _sbr_body(uid_hbm, iid_hbm, w_hbm, ue_hbm, ie_hbm, se_hbm, mean_hbm,
              out_hbm,
              uidx_v, iidx_v, mean_v, U_v, I_v, S_v, W_v, prod_v, out_v, sem):
    wid = lax.axis_index("s") * NC + lax.axis_index("c")
    base = wid * PER_W

    pltpu.sync_copy(uid_hbm.at[wid], uidx_v)
    pltpu.sync_copy(iid_hbm.at[wid], iidx_v)
    pltpu.sync_copy(mean_hbm, mean_v)

    lane_iota = lax.iota(jnp.int32, LANES)
    mean_vec = mean_v[...]

    for c in range(CHUNKS):
        row0 = base + c * CHUNK
        cps = [
            pltpu.make_async_copy(ue_hbm.at[uidx_v.at[c]], U_v, sem),
            pltpu.make_async_copy(ie_hbm.at[iidx_v.at[c]], I_v, sem),
            pltpu.make_async_copy(se_hbm.at[iidx_v.at[c]], S_v, sem),
            pltpu.make_async_copy(w_hbm.at[pl.ds(row0, CHUNK), :], W_v, sem),
        ]
        for cp in cps:
            cp.start()
        for cp in cps:
            cp.wait()

        def group_body(g, _, c=c):
            for j in range(LANES):
                e = g * LANES + j
                p = U_v[e, pl.ds(0, LANES)] * I_v[e, pl.ds(0, LANES)]
                for k in range(1, UI_K):
                    p += U_v[e, pl.ds(k * LANES, LANES)] * \
                         I_v[e, pl.ds(k * LANES, LANES)]
                for k in range(SW_K):
                    p += S_v[e, pl.ds(k * LANES, LANES)] * \
                         W_v[e, pl.ds(k * LANES, LANES)]
                plsc.store_scatter(
                    prod_v, [lane_iota, jnp.full((LANES,), j, jnp.int32)], p)
            acc = mean_vec + prod_v[0, :]
            for r in range(1, LANES):
                acc += prod_v[r, :]
            out_v[pl.ds(c * CHUNK + g * LANES, LANES)] = acc
            return 0

        lax.fori_loop(0, CHUNK // LANES, group_body, 0)

    pltpu.sync_copy(out_v, out_hbm.at[pl.ds(base, PER_W)])


def kernel(u_id, i_id, UserShadow, user_emb, user_bias, item_emb, item_bias,
           shadow_i_emb, mean):
    n_u = user_emb.shape[0]
    n_i = item_emb.shape[0]
    ones_u = jnp.ones((n_u, 1), jnp.float32)
    ones_i = jnp.ones((n_i, 1), jnp.float32)
    zpad_u = jnp.zeros((n_u, 128 - EMB - 2), jnp.float32)
    zpad_i = jnp.zeros((n_i, 128 - EMB - 2), jnp.float32)
    ue_aug = jnp.concatenate([user_emb, user_bias, ones_u, zpad_u], axis=1)
    ie_aug = jnp.concatenate([item_emb, ones_i, item_bias, zpad_i], axis=1)
    se_aug = jnp.concatenate(
        [shadow_i_emb, jnp.zeros((n_i, 128 - EMB), jnp.float32)], axis=1)
    w_pad = jnp.concatenate(
        [UserShadow, jnp.zeros((B, 128 - EMB), jnp.float32)], axis=1)
    uid3 = u_id.reshape(NW, CHUNKS, CHUNK)
    iid3 = i_id.reshape(NW, CHUNKS, CHUNK)
    mean16 = jnp.broadcast_to(mean, (LANES,))

    mesh = plsc.VectorSubcoreMesh(core_axis_name="c", subcore_axis_name="s")
    run = pl.kernel(
        _sbr_body,
        out_type=jax.ShapeDtypeStruct((B,), jnp.float32),
        mesh=mesh,
        compiler_params=pltpu.CompilerParams(
            needs_layout_passes=False, use_tc_tiling_on_sc=True),
        scratch_types=[
            pltpu.VMEM((CHUNKS, CHUNK), jnp.int32),
            pltpu.VMEM((CHUNKS, CHUNK), jnp.int32),
            pltpu.VMEM((LANES,), jnp.float32),
            pltpu.VMEM((CHUNK, 128), jnp.float32),
            pltpu.VMEM((CHUNK, 128), jnp.float32),
            pltpu.VMEM((CHUNK, 128), jnp.float32),
            pltpu.VMEM((CHUNK, 128), jnp.float32),
            pltpu.VMEM((LANES, LANES), jnp.float32),
            pltpu.VMEM((PER_W,), jnp.float32),
            pltpu.SemaphoreType.DMA,
        ],
    )
    return run(uid3, iid3, w_pad, ue_aug, ie_aug, se_aug, mean16)


# speedup vs baseline: 1.9347x; 1.9347x over previous
import jax
import jax.numpy as jnp
from jax import lax
from jax.experimental import pallas as pl
from jax.experimental.pallas import tpu as pltpu
from jax.experimental.pallas import tpu_sc as plsc

B = 16384
EMB = 64
NC = 2
NS = 16
NW = NC * NS
CHUNK = 128
CHUNKS = B // NW // CHUNK
PER_W = CHUNKS * CHUNK
LANES = 16
KV = EMB // LANES


def _sbr_body(uid_hbm, iid_hbm, w_hbm, ue_hbm, ub_hbm, ie_hbm, ib_hbm,
              se_hbm, mean_hbm, out_hbm,
              uidx_v, iidx_v, bu_v, bi_v, mean_v,
              U_v, I_v, S_v, W_v, prod_v, out_v, sem, rsem):
    wid = lax.axis_index("s") * NC + lax.axis_index("c")
    base = wid * PER_W

    pltpu.sync_copy(uid_hbm.at[pl.ds(base, PER_W)], uidx_v)
    pltpu.sync_copy(iid_hbm.at[pl.ds(base, PER_W)], iidx_v)
    pltpu.sync_copy(mean_hbm, mean_v)

    bias_cps = []
    for c in range(CHUNKS):
        bias_cps.append(pltpu.make_async_copy(
            ub_hbm.at[uidx_v.at[pl.ds(c * CHUNK, CHUNK)]],
            bu_v.at[pl.ds(c * CHUNK, CHUNK)], sem))
        bias_cps.append(pltpu.make_async_copy(
            ib_hbm.at[iidx_v.at[pl.ds(c * CHUNK, CHUNK)]],
            bi_v.at[pl.ds(c * CHUNK, CHUNK)], sem))
    for cp in bias_cps:
        cp.start()
    for cp in bias_cps:
        cp.wait()

    lane_iota = lax.iota(jnp.int32, LANES)
    mean_vec = mean_v[...]

    for c in range(CHUNKS):
        row0 = base + c * CHUNK

        def fire_rows(g, _, c=c):
            uvec = uidx_v[pl.ds(c * CHUNK + g * LANES, LANES)]
            ivec = iidx_v[pl.ds(c * CHUNK + g * LANES, LANES)]
            for j in range(LANES):
                ru = jnp.sum(jnp.where(lane_iota == j, uvec, 0))
                ri = jnp.sum(jnp.where(lane_iota == j, ivec, 0))
                e = g * LANES + j
                pltpu.make_async_copy(ue_hbm.at[ru], U_v.at[e], rsem).start()
                pltpu.make_async_copy(ie_hbm.at[ri], I_v.at[e], rsem).start()
                pltpu.make_async_copy(se_hbm.at[ri], S_v.at[e], rsem).start()
            return 0

        lax.fori_loop(0, CHUNK // LANES, fire_rows, 0)
        pltpu.async_copy(w_hbm.at[pl.ds(row0, CHUNK), :], W_v, sem).wait()
        pltpu.make_async_copy(ue_hbm.at[pl.ds(0, CHUNK), :], U_v, rsem).wait()
        pltpu.make_async_copy(ie_hbm.at[pl.ds(0, CHUNK), :], I_v, rsem).wait()
        pltpu.make_async_copy(se_hbm.at[pl.ds(0, CHUNK), :], S_v, rsem).wait()

        def group_body(g, _, c=c):
            for j in range(LANES):
                e = g * LANES + j
                p = U_v[e, pl.ds(0, LANES)] * I_v[e, pl.ds(0, LANES)]
                for k in range(1, KV):
                    p += U_v[e, pl.ds(k * LANES, LANES)] * \
                         I_v[e, pl.ds(k * LANES, LANES)]
                for k in range(KV):
                    p += S_v[e, pl.ds(k * LANES, LANES)] * \
                         W_v[e, pl.ds(k * LANES, LANES)]
                plsc.store_scatter(
                    prod_v, [lane_iota, jnp.full((LANES,), j, jnp.int32)], p)
            acc = mean_vec + prod_v[0, :]
            for r in range(1, LANES):
                acc += prod_v[r, :]
            acc += bu_v[pl.ds(c * CHUNK + g * LANES, LANES)]
            acc += bi_v[pl.ds(c * CHUNK + g * LANES, LANES)]
            out_v[pl.ds(c * CHUNK + g * LANES, LANES)] = acc
            return 0

        lax.fori_loop(0, CHUNK // LANES, group_body, 0)

    pltpu.sync_copy(out_v, out_hbm.at[pl.ds(base, PER_W)])


def kernel(u_id, i_id, UserShadow, user_emb, user_bias, item_emb, item_bias,
           shadow_i_emb, mean):
    ub_flat = user_bias.reshape(-1)
    ib_flat = item_bias.reshape(-1)
    mean16 = jnp.broadcast_to(mean, (LANES,))

    mesh = plsc.VectorSubcoreMesh(core_axis_name="c", subcore_axis_name="s")
    run = pl.kernel(
        _sbr_body,
        out_type=jax.ShapeDtypeStruct((B,), jnp.float32),
        mesh=mesh,
        compiler_params=pltpu.CompilerParams(
            needs_layout_passes=False, use_tc_tiling_on_sc=True),
        scratch_types=[
            pltpu.VMEM((PER_W,), jnp.int32),
            pltpu.VMEM((PER_W,), jnp.int32),
            pltpu.VMEM((PER_W,), jnp.float32),
            pltpu.VMEM((PER_W,), jnp.float32),
            pltpu.VMEM((LANES,), jnp.float32),
            pltpu.VMEM((CHUNK, EMB), jnp.float32),
            pltpu.VMEM((CHUNK, EMB), jnp.float32),
            pltpu.VMEM((CHUNK, EMB), jnp.float32),
            pltpu.VMEM((CHUNK, EMB), jnp.float32),
            pltpu.VMEM((LANES, LANES), jnp.float32),
            pltpu.VMEM((PER_W,), jnp.float32),
            pltpu.SemaphoreType.DMA,
            pltpu.SemaphoreType.DMA,
        ],
    )
    return run(u_id, i_id, UserShadow, user_emb, ub_flat, item_emb, ib_flat,
               shadow_i_emb, mean16)


# speedup vs baseline: 2.0701x; 1.0700x over previous
import jax
import jax.numpy as jnp
from jax import lax
from jax.experimental import pallas as pl
from jax.experimental.pallas import tpu as pltpu
from jax.experimental.pallas import tpu_sc as plsc

B = 16384
EMB = 64
NC = 2
NS = 16
NW = NC * NS
CHUNK = 128
CHUNKS = B // NW // CHUNK
PER_W = CHUNKS * CHUNK
LANES = 16
KV = EMB // LANES

_COMPILER_PARAMS = pltpu.CompilerParams(
    needs_layout_passes=False, use_tc_tiling_on_sc=True)


def _ui_body(uid_hbm, iid_hbm, ue_hbm, ub_hbm, ie_hbm, ib_hbm, mean_hbm,
             out_hbm,
             uidx_v, iidx_v, bu_v, bi_v, mean_v,
             U_v, I_v, prod_v, out_v, sem, rsem):
    wid = lax.axis_index("s") * NC + lax.axis_index("c")
    base = wid * PER_W

    pltpu.sync_copy(uid_hbm.at[pl.ds(base, PER_W)], uidx_v)
    pltpu.sync_copy(iid_hbm.at[pl.ds(base, PER_W)], iidx_v)
    pltpu.sync_copy(mean_hbm, mean_v)

    bias_cps = []
    for c in range(CHUNKS):
        bias_cps.append(pltpu.make_async_copy(
            ub_hbm.at[uidx_v.at[pl.ds(c * CHUNK, CHUNK)]],
            bu_v.at[pl.ds(c * CHUNK, CHUNK)], sem))
        bias_cps.append(pltpu.make_async_copy(
            ib_hbm.at[iidx_v.at[pl.ds(c * CHUNK, CHUNK)]],
            bi_v.at[pl.ds(c * CHUNK, CHUNK)], sem))
    for cp in bias_cps:
        cp.start()
    for cp in bias_cps:
        cp.wait()

    lane_iota = lax.iota(jnp.int32, LANES)
    mean_vec = mean_v[...]

    for c in range(CHUNKS):
        def fire_rows(g, _, c=c):
            uvec = uidx_v[pl.ds(c * CHUNK + g * LANES, LANES)]
            ivec = iidx_v[pl.ds(c * CHUNK + g * LANES, LANES)]
            for j in range(LANES):
                ru = jnp.sum(jnp.where(lane_iota == j, uvec, 0))
                ri = jnp.sum(jnp.where(lane_iota == j, ivec, 0))
                e = g * LANES + j
                pltpu.make_async_copy(ue_hbm.at[ru], U_v.at[e], rsem).start()
                pltpu.make_async_copy(ie_hbm.at[ri], I_v.at[e], rsem).start()
            return 0

        lax.fori_loop(0, CHUNK // LANES, fire_rows, 0)
        pltpu.make_async_copy(ue_hbm.at[pl.ds(0, CHUNK), :], U_v, rsem).wait()
        pltpu.make_async_copy(ie_hbm.at[pl.ds(0, CHUNK), :], I_v, rsem).wait()

        def group_body(g, _, c=c):
            for j in range(LANES):
                e = g * LANES + j
                p = U_v[e, pl.ds(0, LANES)] * I_v[e, pl.ds(0, LANES)]
                for k in range(1, KV):
                    p += U_v[e, pl.ds(k * LANES, LANES)] * \
                         I_v[e, pl.ds(k * LANES, LANES)]
                plsc.store_scatter(
                    prod_v, [lane_iota, jnp.full((LANES,), j, jnp.int32)], p)
            acc = mean_vec + prod_v[0, :]
            for r in range(1, LANES):
                acc += prod_v[r, :]
            acc += bu_v[pl.ds(c * CHUNK + g * LANES, LANES)]
            acc += bi_v[pl.ds(c * CHUNK + g * LANES, LANES)]
            out_v[pl.ds(c * CHUNK + g * LANES, LANES)] = acc
            return 0

        lax.fori_loop(0, CHUNK // LANES, group_body, 0)

    pltpu.sync_copy(out_v, out_hbm.at[pl.ds(base, PER_W)])


def _sw_body(iid_hbm, w_hbm, se_hbm, in1_hbm, out_hbm,
             iidx_v, in1_v, S_v, W_v, prod_v, out_v, sem, rsem):
    wid = lax.axis_index("s") * NC + lax.axis_index("c")
    base = wid * PER_W

    pltpu.sync_copy(iid_hbm.at[pl.ds(base, PER_W)], iidx_v)
    pltpu.sync_copy(in1_hbm.at[pl.ds(base, PER_W)], in1_v)

    lane_iota = lax.iota(jnp.int32, LANES)

    for c in range(CHUNKS):
        row0 = base + c * CHUNK

        def fire_rows(g, _, c=c):
            ivec = iidx_v[pl.ds(c * CHUNK + g * LANES, LANES)]
            for j in range(LANES):
                ri = jnp.sum(jnp.where(lane_iota == j, ivec, 0))
                e = g * LANES + j
                pltpu.make_async_copy(se_hbm.at[ri], S_v.at[e], rsem).start()
            return 0

        lax.fori_loop(0, CHUNK // LANES, fire_rows, 0)
        pltpu.async_copy(w_hbm.at[pl.ds(row0, CHUNK), :], W_v, sem).wait()
        pltpu.make_async_copy(se_hbm.at[pl.ds(0, CHUNK), :], S_v, rsem).wait()

        def group_body(g, _, c=c):
            for j in range(LANES):
                e = g * LANES + j
                p = S_v[e, pl.ds(0, LANES)] * W_v[e, pl.ds(0, LANES)]
                for k in range(1, KV):
                    p += S_v[e, pl.ds(k * LANES, LANES)] * \
                         W_v[e, pl.ds(k * LANES, LANES)]
                plsc.store_scatter(
                    prod_v, [lane_iota, jnp.full((LANES,), j, jnp.int32)], p)
            acc = in1_v[pl.ds(c * CHUNK + g * LANES, LANES)] + prod_v[0, :]
            for r in range(1, LANES):
                acc += prod_v[r, :]
            out_v[pl.ds(c * CHUNK + g * LANES, LANES)] = acc
            return 0

        lax.fori_loop(0, CHUNK // LANES, group_body, 0)

    pltpu.sync_copy(out_v, out_hbm.at[pl.ds(base, PER_W)])


def kernel(u_id, i_id, UserShadow, user_emb, user_bias, item_emb, item_bias,
           shadow_i_emb, mean):
    ub_flat = user_bias.reshape(-1)
    ib_flat = item_bias.reshape(-1)
    mean16 = jnp.broadcast_to(mean, (LANES,))

    mesh = plsc.VectorSubcoreMesh(core_axis_name="c", subcore_axis_name="s")
    run_ui = pl.kernel(
        _ui_body,
        out_type=jax.ShapeDtypeStruct((B,), jnp.float32),
        mesh=mesh,
        compiler_params=_COMPILER_PARAMS,
        scratch_types=[
            pltpu.VMEM((PER_W,), jnp.int32),
            pltpu.VMEM((PER_W,), jnp.int32),
            pltpu.VMEM((PER_W,), jnp.float32),
            pltpu.VMEM((PER_W,), jnp.float32),
            pltpu.VMEM((LANES,), jnp.float32),
            pltpu.VMEM((CHUNK, EMB), jnp.float32),
            pltpu.VMEM((CHUNK, EMB), jnp.float32),
            pltpu.VMEM((LANES, LANES), jnp.float32),
            pltpu.VMEM((PER_W,), jnp.float32),
            pltpu.SemaphoreType.DMA,
            pltpu.SemaphoreType.DMA,
        ],
    )
    out1 = run_ui(u_id, i_id, user_emb, ub_flat, item_emb, ib_flat, mean16)

    run_sw = pl.kernel(
        _sw_body,
        out_type=jax.ShapeDtypeStruct((B,), jnp.float32),
        mesh=mesh,
        compiler_params=_COMPILER_PARAMS,
        scratch_types=[
            pltpu.VMEM((PER_W,), jnp.int32),
            pltpu.VMEM((PER_W,), jnp.float32),
            pltpu.VMEM((CHUNK, EMB), jnp.float32),
            pltpu.VMEM((CHUNK, EMB), jnp.float32),
            pltpu.VMEM((LANES, LANES), jnp.float32),
            pltpu.VMEM((PER_W,), jnp.float32),
            pltpu.SemaphoreType.DMA,
            pltpu.SemaphoreType.DMA,
        ],
    )
    return run_sw(i_id, UserShadow, shadow_i_emb, out1)


# speedup vs baseline: 2.0909x; 1.0100x over previous
import jax
import jax.numpy as jnp
from jax import lax
from jax.experimental import pallas as pl
from jax.experimental.pallas import tpu as pltpu
from jax.experimental.pallas import tpu_sc as plsc

B = 16384
EMB = 64
NC = 2
NS = 16
NW = NC * NS
CHUNK = 128
CHUNKS = B // NW // CHUNK
PER_W = CHUNKS * CHUNK
LANES = 16
KV = EMB // LANES

_COMPILER_PARAMS = pltpu.CompilerParams(
    needs_layout_passes=False, use_tc_tiling_on_sc=True)


def _ui_body(uid_hbm, iid_hbm, ue_hbm, ub_hbm, ie_hbm, ib_hbm, mean_hbm,
             out_hbm,
             uidx_v, iidx_v, bu_v, bi_v, mean_v,
             U_v, I_v, prod_v, out_v, sem, rsem):
    wid = lax.axis_index("s") * NC + lax.axis_index("c")
    base = wid * PER_W

    pltpu.sync_copy(uid_hbm.at[pl.ds(base, PER_W)], uidx_v)
    pltpu.sync_copy(iid_hbm.at[pl.ds(base, PER_W)], iidx_v)
    pltpu.sync_copy(mean_hbm, mean_v)

    bias_cps = []
    for c in range(CHUNKS):
        bias_cps.append(pltpu.make_async_copy(
            ub_hbm.at[uidx_v.at[pl.ds(c * CHUNK, CHUNK)]],
            bu_v.at[pl.ds(c * CHUNK, CHUNK)], sem))
        bias_cps.append(pltpu.make_async_copy(
            ib_hbm.at[iidx_v.at[pl.ds(c * CHUNK, CHUNK)]],
            bi_v.at[pl.ds(c * CHUNK, CHUNK)], sem))
    for cp in bias_cps:
        cp.start()
    for cp in bias_cps:
        cp.wait()

    lane_iota = lax.iota(jnp.int32, LANES)
    mean_vec = mean_v[...]

    for c in range(CHUNKS):
        def fire_rows(g, _, c=c):
            uvec = uidx_v[pl.ds(c * CHUNK + g * LANES, LANES)]
            ivec = iidx_v[pl.ds(c * CHUNK + g * LANES, LANES)]
            for j in range(LANES):
                ru = jnp.sum(jnp.where(lane_iota == j, uvec, 0))
                ri = jnp.sum(jnp.where(lane_iota == j, ivec, 0))
                e = g * LANES + j
                pltpu.make_async_copy(ue_hbm.at[ru], U_v.at[e], rsem).start()
                pltpu.make_async_copy(ie_hbm.at[ri], I_v.at[e], rsem).start()
            return 0

        lax.fori_loop(0, CHUNK // LANES, fire_rows, 0)
        pltpu.make_async_copy(ue_hbm.at[pl.ds(0, CHUNK), :], U_v, rsem).wait()
        pltpu.make_async_copy(ie_hbm.at[pl.ds(0, CHUNK), :], I_v, rsem).wait()

        def group_body(g, _, c=c):
            for j in range(LANES):
                e = g * LANES + j
                p = U_v[e, pl.ds(0, LANES)] * I_v[e, pl.ds(0, LANES)]
                for k in range(1, KV):
                    p += U_v[e, pl.ds(k * LANES, LANES)] * \
                         I_v[e, pl.ds(k * LANES, LANES)]
                plsc.store_scatter(
                    prod_v, [lane_iota, jnp.full((LANES,), j, jnp.int32)], p)
            acc = mean_vec + prod_v[0, :]
            for r in range(1, LANES):
                acc += prod_v[r, :]
            acc += bu_v[pl.ds(c * CHUNK + g * LANES, LANES)]
            acc += bi_v[pl.ds(c * CHUNK + g * LANES, LANES)]
            out_v[pl.ds(c * CHUNK + g * LANES, LANES)] = acc
            return 0

        lax.fori_loop(0, CHUNK // LANES, group_body, 0)

    pltpu.sync_copy(out_v, out_hbm.at[pl.ds(base, PER_W)])


def _sw_body(iid_hbm, w_hbm, se_hbm, in1_hbm, out_hbm,
             iidx_v, in1_v, S_v, W_v, prod_v, out_v, sem, rsem):
    wid = lax.axis_index("s") * NC + lax.axis_index("c")
    base = wid * PER_W

    pltpu.sync_copy(iid_hbm.at[pl.ds(base, PER_W)], iidx_v)
    pltpu.sync_copy(in1_hbm.at[pl.ds(base, PER_W)], in1_v)

    lane_iota = lax.iota(jnp.int32, LANES)

    def fire_rows(g, _):
        ivec = iidx_v[pl.ds(g * LANES, LANES)]
        for j in range(LANES):
            ri = jnp.sum(jnp.where(lane_iota == j, ivec, 0))
            e = g * LANES + j
            pltpu.make_async_copy(se_hbm.at[ri], S_v.at[e], rsem).start()
        return 0

    lax.fori_loop(0, PER_W // LANES, fire_rows, 0)
    for _ in range(CHUNKS):
        pltpu.make_async_copy(
            se_hbm.at[pl.ds(0, CHUNK), :], S_v.at[pl.ds(0, CHUNK)],
            rsem).wait()

    for c in range(CHUNKS):
        pltpu.async_copy(
            w_hbm.at[pl.ds(base + c * CHUNK, CHUNK), :], W_v, sem).wait()

        def group_body(g, _, c=c):
            for j in range(LANES):
                e = g * LANES + j
                p = S_v[c * CHUNK + e, pl.ds(0, LANES)] * \
                    W_v[e, pl.ds(0, LANES)]
                for k in range(1, KV):
                    p += S_v[c * CHUNK + e, pl.ds(k * LANES, LANES)] * \
                         W_v[e, pl.ds(k * LANES, LANES)]
                plsc.store_scatter(
                    prod_v, [lane_iota, jnp.full((LANES,), j, jnp.int32)], p)
            acc = in1_v[pl.ds(c * CHUNK + g * LANES, LANES)] + prod_v[0, :]
            for r in range(1, LANES):
                acc += prod_v[r, :]
            out_v[pl.ds(c * CHUNK + g * LANES, LANES)] = acc
            return 0

        lax.fori_loop(0, CHUNK // LANES, group_body, 0)

    pltpu.sync_copy(out_v, out_hbm.at[pl.ds(base, PER_W)])


def kernel(u_id, i_id, UserShadow, user_emb, user_bias, item_emb, item_bias,
           shadow_i_emb, mean):
    ub_flat = user_bias.reshape(-1)
    ib_flat = item_bias.reshape(-1)
    mean16 = jnp.broadcast_to(mean, (LANES,))

    mesh = plsc.VectorSubcoreMesh(core_axis_name="c", subcore_axis_name="s")
    run_ui = pl.kernel(
        _ui_body,
        out_type=jax.ShapeDtypeStruct((B,), jnp.float32),
        mesh=mesh,
        compiler_params=_COMPILER_PARAMS,
        scratch_types=[
            pltpu.VMEM((PER_W,), jnp.int32),
            pltpu.VMEM((PER_W,), jnp.int32),
            pltpu.VMEM((PER_W,), jnp.float32),
            pltpu.VMEM((PER_W,), jnp.float32),
            pltpu.VMEM((LANES,), jnp.float32),
            pltpu.VMEM((CHUNK, EMB), jnp.float32),
            pltpu.VMEM((CHUNK, EMB), jnp.float32),
            pltpu.VMEM((LANES, LANES), jnp.float32),
            pltpu.VMEM((PER_W,), jnp.float32),
            pltpu.SemaphoreType.DMA,
            pltpu.SemaphoreType.DMA,
        ],
    )
    out1 = run_ui(u_id, i_id, user_emb, ub_flat, item_emb, ib_flat, mean16)

    run_sw = pl.kernel(
        _sw_body,
        out_type=jax.ShapeDtypeStruct((B,), jnp.float32),
        mesh=mesh,
        compiler_params=_COMPILER_PARAMS,
        scratch_types=[
            pltpu.VMEM((PER_W,), jnp.int32),
            pltpu.VMEM((PER_W,), jnp.float32),
            pltpu.VMEM((PER_W, EMB), jnp.float32),
            pltpu.VMEM((CHUNK, EMB), jnp.float32),
            pltpu.VMEM((LANES, LANES), jnp.float32),
            pltpu.VMEM((PER_W,), jnp.float32),
            pltpu.SemaphoreType.DMA,
            pltpu.SemaphoreType.DMA,
        ],
    )
    return run_sw(i_id, UserShadow, shadow_i_emb, out1)
